# Initial kernel scaffold; baseline (speedup 1.0000x reference)
import jax
import jax.numpy as jnp
from jax.experimental import pallas as pl


def kernel(x):
    raise NotImplementedError("write your pallas kernel here")


# speedup vs baseline: 2.5011x; 2.5011x over previous
import dataclasses
import functools

import jax
import jax.numpy as jnp
from jax import lax
from jax.experimental import pallas as pl
from jax.experimental.pallas import tpu as pltpu
from jax.experimental.pallas import tpu_sc as plsc

_KERNEL_SIZE = 5
_PADDING = 4

_NC = 2
_NS = 16
_LANES = 16


def _compiler_params():
    cp = pltpu.CompilerParams(use_tc_tiling_on_sc=False)
    if "needs_layout_passes" in pltpu.CompilerParams.__dataclass_fields__:
        cp = dataclasses.replace(cp, needs_layout_passes=False)
    return cp


def kernel(x):
    B, F, S = x.shape
    W = (S + 2 * _PADDING) // _KERNEL_SIZE
    WP = ((W + _LANES - 1) // _LANES) * _LANES

    idx = jax.random.randint(
        jax.random.key(1), (B, W), 0, _KERNEL_SIZE, dtype=jnp.int32
    )
    w_ar = jnp.arange(W, dtype=jnp.int32)
    src = idx * W + w_ar[None, :] - _PADDING
    src = jnp.where((src < 0) | (src >= S), S, src)
    src_pad = jnp.full((B, WP), S, jnp.int32).at[:, :W].set(src)

    NW = _NC * _NS
    rows_per_tile = F // NW

    mesh = plsc.VectorSubcoreMesh(core_axis_name="c", subcore_axis_name="s")

    @functools.partial(
        pl.kernel,
        out_type=jax.ShapeDtypeStruct((B, F, W), x.dtype),
        mesh=mesh,
        compiler_params=_compiler_params(),
        scratch_types=[
            pltpu.VMEM((B, WP), jnp.int32),
            pltpu.VMEM((S + _LANES,), jnp.float32),
            pltpu.VMEM((WP,), jnp.float32),
        ],
    )
    def sc_gather_rows(x_hbm, src_hbm, o_hbm, src_v, row_v, out_v):
        wid = lax.axis_index("s") * _NC + lax.axis_index("c")
        pltpu.sync_copy(src_hbm, src_v)
        row_v[pl.ds(S, _LANES)] = jnp.zeros((_LANES,), jnp.float32)
        for b in range(B):
            @pl.loop(0, rows_per_tile)
            def _(r):
                f = wid * rows_per_tile + r
                pltpu.sync_copy(x_hbm.at[b, f], row_v.at[pl.ds(0, S)])

                @pl.loop(0, WP // _LANES)
                def _(j):
                    iv = src_v[b, pl.ds(j * _LANES, _LANES)]
                    out_v[pl.ds(j * _LANES, _LANES)] = plsc.load_gather(
                        row_v, [iv]
                    )

                pltpu.sync_copy(out_v.at[pl.ds(0, W)], o_hbm.at[b, f])

    return sc_gather_rows(x, src_pad)


# speedup vs baseline: 3.2714x; 1.3080x over previous
import dataclasses
import functools

import jax
import jax.numpy as jnp
from jax import lax
from jax.experimental import pallas as pl
from jax.experimental.pallas import tpu as pltpu
from jax.experimental.pallas import tpu_sc as plsc

_KERNEL_SIZE = 5
_PADDING = 4

_NC = 2
_NS = 16
_LANES = 16


def _compiler_params():
    cp = pltpu.CompilerParams(use_tc_tiling_on_sc=False)
    if "needs_layout_passes" in pltpu.CompilerParams.__dataclass_fields__:
        cp = dataclasses.replace(cp, needs_layout_passes=False)
    return cp


def kernel(x):
    B, F, S = x.shape
    W = (S + 2 * _PADDING) // _KERNEL_SIZE
    WP = ((W + _LANES - 1) // _LANES) * _LANES

    idx = jax.random.randint(
        jax.random.key(1), (B, W), 0, _KERNEL_SIZE, dtype=jnp.int32
    )
    w_ar = jnp.arange(W, dtype=jnp.int32)
    src = idx * W + w_ar[None, :] - _PADDING
    src = jnp.where((src < 0) | (src >= S), S, src)
    src_pad = jnp.full((B, WP), S, jnp.int32).at[:, :W].set(src)

    NW = _NC * _NS
    R = B * F
    rows_per_tile = R // NW
    x2 = x.reshape(R, S)

    mesh = plsc.VectorSubcoreMesh(core_axis_name="c", subcore_axis_name="s")

    @functools.partial(
        pl.kernel,
        out_type=jax.ShapeDtypeStruct((R, W), x.dtype),
        mesh=mesh,
        compiler_params=_compiler_params(),
        scratch_types=[
            pltpu.VMEM((B, WP), jnp.int32),
            pltpu.VMEM((S + _LANES,), jnp.float32),
            pltpu.VMEM((S + _LANES,), jnp.float32),
            pltpu.VMEM((WP,), jnp.float32),
            pltpu.VMEM((WP,), jnp.float32),
            pltpu.SemaphoreType.DMA,
            pltpu.SemaphoreType.DMA,
            pltpu.SemaphoreType.DMA,
            pltpu.SemaphoreType.DMA,
        ],
    )
    def sc_gather_rows(x_hbm, src_hbm, o_hbm, src_v, ib0, ib1, ob0, ob1,
                       isem0, isem1, osem0, osem1):
        wid = lax.axis_index("s") * _NC + lax.axis_index("c")
        base = wid * rows_per_tile
        bt = base // F
        pltpu.sync_copy(src_hbm, src_v)
        zeros = jnp.zeros((_LANES,), jnp.float32)
        ib0[pl.ds(S, _LANES)] = zeros
        ib1[pl.ds(S, _LANES)] = zeros

        def in_copy(t, buf, sem):
            return pltpu.make_async_copy(
                x_hbm.at[base + t], buf.at[pl.ds(0, S)], sem
            )

        def out_copy(t, buf, sem):
            return pltpu.make_async_copy(
                buf.at[pl.ds(0, W)], o_hbm.at[base + t], sem
            )

        def gather_row(ibuf, obuf):
            @pl.loop(0, WP // _LANES)
            def _(j):
                iv = src_v[bt, pl.ds(j * _LANES, _LANES)]
                obuf[pl.ds(j * _LANES, _LANES)] = plsc.load_gather(ibuf, [iv])

        in_copy(0, ib0, isem0).start()
        in_copy(1, ib1, isem1).start()

        @pl.loop(0, rows_per_tile, step=2)
        def _(t):
            in_copy(t, ib0, isem0).wait()

            @pl.when(t >= 2)
            def _():
                out_copy(t - 2, ob0, osem0).wait()

            gather_row(ib0, ob0)
            out_copy(t, ob0, osem0).start()

            @pl.when(t < rows_per_tile - 2)
            def _():
                in_copy(t + 2, ib0, isem0).start()

            in_copy(t + 1, ib1, isem1).wait()

            @pl.when(t >= 2)
            def _():
                out_copy(t - 1, ob1, osem1).wait()

            gather_row(ib1, ob1)
            out_copy(t + 1, ob1, osem1).start()

            @pl.when(t < rows_per_tile - 3)
            def _():
                in_copy(t + 3, ib1, isem1).start()

        out_copy(rows_per_tile - 2, ob0, osem0).wait()
        out_copy(rows_per_tile - 1, ob1, osem1).wait()

    return sc_gather_rows(x2, src_pad).reshape(B, F, W)


# speedup vs baseline: 3.8701x; 1.1830x over previous
import dataclasses
import functools

import jax
import jax.numpy as jnp
from jax import lax
from jax.experimental import pallas as pl
from jax.experimental.pallas import tpu as pltpu
from jax.experimental.pallas import tpu_sc as plsc

_KERNEL_SIZE = 5
_PADDING = 4

_NC = 2
_NS = 16
_LANES = 16


def _compiler_params():
    cp = pltpu.CompilerParams(use_tc_tiling_on_sc=False)
    if "needs_layout_passes" in pltpu.CompilerParams.__dataclass_fields__:
        cp = dataclasses.replace(cp, needs_layout_passes=False)
    return cp


def kernel(x):
    B, F, S = x.shape
    W = (S + 2 * _PADDING) // _KERNEL_SIZE
    UNROLL = 8
    CHUNK = _LANES * UNROLL
    WP = ((W + CHUNK - 1) // CHUNK) * CHUNK

    idx = jax.random.randint(
        jax.random.key(1), (B, W), 0, _KERNEL_SIZE, dtype=jnp.int32
    )
    w_ar = jnp.arange(W, dtype=jnp.int32)
    src = idx * W + w_ar[None, :] - _PADDING
    src = jnp.where((src < 0) | (src >= S), S, src)
    src_pad = jnp.full((B, WP), S, jnp.int32).at[:, :W].set(src)

    NW = _NC * _NS
    R = B * F
    rows_per_tile = R // NW
    x2 = x.reshape(R, S)

    mesh = plsc.VectorSubcoreMesh(core_axis_name="c", subcore_axis_name="s")

    @functools.partial(
        pl.kernel,
        out_type=jax.ShapeDtypeStruct((R, W), x.dtype),
        mesh=mesh,
        compiler_params=_compiler_params(),
        scratch_types=[
            pltpu.VMEM((B, WP), jnp.int32),
            pltpu.VMEM((S + _LANES,), jnp.float32),
            pltpu.VMEM((S + _LANES,), jnp.float32),
            pltpu.VMEM((WP,), jnp.float32),
            pltpu.VMEM((WP,), jnp.float32),
            pltpu.SemaphoreType.DMA,
            pltpu.SemaphoreType.DMA,
            pltpu.SemaphoreType.DMA,
            pltpu.SemaphoreType.DMA,
        ],
    )
    def sc_gather_rows(x_hbm, src_hbm, o_hbm, src_v, ib0, ib1, ob0, ob1,
                       isem0, isem1, osem0, osem1):
        wid = lax.axis_index("s") * _NC + lax.axis_index("c")
        base = wid * rows_per_tile
        bt = base // F
        pltpu.sync_copy(src_hbm, src_v)
        zeros = jnp.zeros((_LANES,), jnp.float32)
        ib0[pl.ds(S, _LANES)] = zeros
        ib1[pl.ds(S, _LANES)] = zeros

        def in_copy(t, buf, sem):
            return pltpu.make_async_copy(
                x_hbm.at[base + t], buf.at[pl.ds(0, S)], sem
            )

        def out_copy(t, buf, sem):
            return pltpu.make_async_copy(
                buf.at[pl.ds(0, W)], o_hbm.at[base + t], sem
            )

        def gather_row(ibuf, obuf):
            @pl.loop(0, WP // CHUNK)
            def _(j):
                base_w = j * CHUNK
                ivs = [
                    src_v[bt, pl.ds(base_w + u * _LANES, _LANES)]
                    for u in range(UNROLL)
                ]
                gs = [plsc.load_gather(ibuf, [iv]) for iv in ivs]
                for u in range(UNROLL):
                    obuf[pl.ds(base_w + u * _LANES, _LANES)] = gs[u]

        in_copy(0, ib0, isem0).start()
        in_copy(1, ib1, isem1).start()

        @pl.loop(0, rows_per_tile, step=2)
        def _(t):
            in_copy(t, ib0, isem0).wait()

            @pl.when(t >= 2)
            def _():
                out_copy(t - 2, ob0, osem0).wait()

            gather_row(ib0, ob0)
            out_copy(t, ob0, osem0).start()

            @pl.when(t < rows_per_tile - 2)
            def _():
                in_copy(t + 2, ib0, isem0).start()

            in_copy(t + 1, ib1, isem1).wait()

            @pl.when(t >= 2)
            def _():
                out_copy(t - 1, ob1, osem1).wait()

            gather_row(ib1, ob1)
            out_copy(t + 1, ob1, osem1).start()

            @pl.when(t < rows_per_tile - 3)
            def _():
                in_copy(t + 3, ib1, isem1).start()

        out_copy(rows_per_tile - 2, ob0, osem0).wait()
        out_copy(rows_per_tile - 1, ob1, osem1).wait()

    return sc_gather_rows(x2, src_pad).reshape(B, F, W)


# speedup vs baseline: 5.2168x; 1.3480x over previous
import dataclasses
import functools

import jax
import jax.numpy as jnp
from jax import lax
from jax.experimental import pallas as pl
from jax.experimental.pallas import tpu as pltpu
from jax.experimental.pallas import tpu_sc as plsc

_KERNEL_SIZE = 5
_PADDING = 4

_NC = 2
_NS = 16
_LANES = 16


def _compiler_params():
    cp = pltpu.CompilerParams(use_tc_tiling_on_sc=True)
    if "needs_layout_passes" in pltpu.CompilerParams.__dataclass_fields__:
        cp = dataclasses.replace(cp, needs_layout_passes=False)
    return cp


def kernel(x):
    B, F, S = x.shape
    W = (S + 2 * _PADDING) // _KERNEL_SIZE
    WP = ((W + 127) // 128) * 128
    CT = WP // 128

    idx = jax.random.randint(
        jax.random.key(1), (B, W), 0, _KERNEL_SIZE, dtype=jnp.int32
    )
    w_ar = jnp.arange(W, dtype=jnp.int32)
    src = idx * W + w_ar[None, :] - _PADDING
    valid = (src >= 0) & (src < S)
    src_pad = jnp.zeros((B, WP), jnp.int32).at[:, :W].set(
        jnp.where(valid, src, 0)
    )
    mask = jnp.zeros((B, WP), jnp.float32).at[:, :W].set(
        valid.astype(jnp.float32)
    )

    NW = _NC * _NS
    R = B * F
    NBLK = R // 8
    blocks_per_tile = NBLK // NW
    x3 = x.reshape(NBLK, 8, S)

    mesh = plsc.VectorSubcoreMesh(core_axis_name="c", subcore_axis_name="s")

    @functools.partial(
        pl.kernel,
        out_type=jax.ShapeDtypeStruct((NBLK, CT, 8, 128), x.dtype),
        mesh=mesh,
        compiler_params=_compiler_params(),
        scratch_types=[
            pltpu.VMEM((B, WP), jnp.int32),
            pltpu.VMEM((B, WP), jnp.float32),
            pltpu.VMEM((8, S), jnp.float32),
            pltpu.VMEM((CT, 8, 128), jnp.float32),
            pltpu.VMEM((CT, 8, 128), jnp.float32),
            pltpu.SemaphoreType.DMA,
            pltpu.SemaphoreType.DMA,
        ],
    )
    def sc_gather_blocks(x_hbm, src_hbm, msk_hbm, o_hbm, src_v, msk_v, ib,
                         ob0, ob1, osem0, osem1):
        wid = lax.axis_index("s") * _NC + lax.axis_index("c")
        base = wid * blocks_per_tile
        bt = base // (NBLK // B)
        pltpu.sync_copy(src_hbm, src_v)
        pltpu.sync_copy(msk_hbm, msk_v)

        def out_copy(t, obuf, sem):
            return pltpu.make_async_copy(obuf, o_hbm.at[base + t], sem)

        def do_block(t, obuf):
            pltpu.sync_copy(x_hbm.at[base + t], ib)

            @pl.loop(0, CT)
            def _(c):
                for u in range(8):
                    iv = src_v[bt, pl.ds(c * 128 + u * 16, _LANES)]
                    mv = msk_v[bt, pl.ds(c * 128 + u * 16, _LANES)]
                    for fs in range(8):
                        rowv = jnp.full((_LANES,), fs, jnp.int32)
                        g = plsc.load_gather(ib, [rowv, iv])
                        obuf[c, fs, pl.ds(u * 16, _LANES)] = g * mv

        @pl.loop(0, blocks_per_tile, step=2)
        def _(t):
            @pl.when(t >= 2)
            def _():
                out_copy(t - 2, ob0, osem0).wait()

            do_block(t, ob0)
            out_copy(t, ob0, osem0).start()

            @pl.when(t >= 2)
            def _():
                out_copy(t - 1, ob1, osem1).wait()

            do_block(t + 1, ob1)
            out_copy(t + 1, ob1, osem1).start()

        out_copy(blocks_per_tile - 2, ob0, osem0).wait()
        out_copy(blocks_per_tile - 1, ob1, osem1).wait()

    y = sc_gather_blocks(x3, src_pad, mask)
    y = y.transpose(0, 2, 1, 3).reshape(R, WP)[:, :W]
    return y.reshape(B, F, W)


# speedup vs baseline: 8.8454x; 1.6956x over previous
import dataclasses
import functools

import jax
import jax.numpy as jnp
from jax import lax
from jax.experimental import pallas as pl
from jax.experimental.pallas import tpu as pltpu
from jax.experimental.pallas import tpu_sc as plsc

_KERNEL_SIZE = 5
_PADDING = 4

_NC = 2
_NS = 16
_LANES = 16

_SC_ROWS = 1024
_TC_BLK = 256


def _compiler_params():
    cp = pltpu.CompilerParams(use_tc_tiling_on_sc=True)
    if "needs_layout_passes" in pltpu.CompilerParams.__dataclass_fields__:
        cp = dataclasses.replace(cp, needs_layout_passes=False)
    return cp


def kernel(x):
    B, F, S = x.shape
    W = (S + 2 * _PADDING) // _KERNEL_SIZE
    WP = ((W + 127) // 128) * 128
    CT = WP // 128

    idx = jax.random.randint(
        jax.random.key(1), (B, W), 0, _KERNEL_SIZE, dtype=jnp.int32
    )
    w_ar = jnp.arange(W, dtype=jnp.int32)
    src = idx * W + w_ar[None, :] - _PADDING
    valid = (src >= 0) & (src < S)

    src_pad = jnp.zeros((B, WP), jnp.int32).at[:, :W].set(
        jnp.where(valid, src, 0)
    )
    mask = jnp.zeros((B, WP), jnp.float32).at[:, :W].set(
        valid.astype(jnp.float32)
    )

    sel = (idx[:, None, :] == jnp.arange(_KERNEL_SIZE)[None, :, None])
    selm = jnp.zeros((B, _KERNEL_SIZE, WP), jnp.float32)
    selm = selm.at[:, :, :W].set(
        (sel & valid[:, None, :]).astype(jnp.float32)
    )

    NW = _NC * _NS
    R = B * F
    NBLK = R // 8
    TC_ROWS = R - _SC_ROWS
    SC_BLKS = _SC_ROWS // 8
    SC_OFF_BLK = TC_ROWS // 8
    blocks_per_tile = SC_BLKS // NW
    x3 = x.reshape(NBLK, 8, S)
    x2 = x.reshape(R, S)

    mesh = plsc.VectorSubcoreMesh(core_axis_name="c", subcore_axis_name="s")

    @functools.partial(
        pl.kernel,
        out_type=jax.ShapeDtypeStruct((SC_BLKS, CT, 8, 128), x.dtype),
        mesh=mesh,
        compiler_params=_compiler_params(),
        scratch_types=[
            pltpu.VMEM((B, WP), jnp.int32),
            pltpu.VMEM((B, WP), jnp.float32),
            pltpu.VMEM((8, S), jnp.float32),
            pltpu.VMEM((CT, 8, 128), jnp.float32),
            pltpu.VMEM((CT, 8, 128), jnp.float32),
            pltpu.SemaphoreType.DMA,
            pltpu.SemaphoreType.DMA,
        ],
    )
    def sc_gather_blocks(x_hbm, src_hbm, msk_hbm, o_hbm, src_v, msk_v, ib,
                         ob0, ob1, osem0, osem1):
        wid = lax.axis_index("s") * _NC + lax.axis_index("c")
        base = SC_OFF_BLK + wid * blocks_per_tile
        bt = (base * 8) // F
        pltpu.sync_copy(src_hbm, src_v)
        pltpu.sync_copy(msk_hbm, msk_v)

        def out_copy(t, obuf, sem):
            return pltpu.make_async_copy(
                obuf, o_hbm.at[base - SC_OFF_BLK + t], sem
            )

        def do_block(t, obuf):
            pltpu.sync_copy(x_hbm.at[base + t], ib)

            @pl.loop(0, CT)
            def _(c):
                for u in range(8):
                    iv = src_v[bt, pl.ds(c * 128 + u * 16, _LANES)]
                    mv = msk_v[bt, pl.ds(c * 128 + u * 16, _LANES)]
                    for fs in range(8):
                        rowv = jnp.full((_LANES,), fs, jnp.int32)
                        g = plsc.load_gather(ib, [rowv, iv])
                        obuf[c, fs, pl.ds(u * 16, _LANES)] = g * mv

        @pl.loop(0, blocks_per_tile, step=2)
        def _(t):
            @pl.when(t >= 2)
            def _():
                out_copy(t - 2, ob0, osem0).wait()

            do_block(t, ob0)
            out_copy(t, ob0, osem0).start()

            @pl.when(t >= 2)
            def _():
                out_copy(t - 1, ob1, osem1).wait()

            do_block(t + 1, ob1)
            out_copy(t + 1, ob1, osem1).start()

        out_copy(blocks_per_tile - 2, ob0, osem0).wait()
        out_copy(blocks_per_tile - 1, ob1, osem1).wait()

    def tc_select_body(x_ref, m_ref, o_ref):
        xb = x_ref[...]
        z4 = jnp.zeros((_TC_BLK, _PADDING), jnp.float32)
        acc = None
        for k in range(_KERNEL_SIZE):
            lo = k * W - _PADDING
            if lo < 0:
                cand = jnp.concatenate([z4, xb[:, : W + lo]], axis=1)
            elif lo + W > S:
                cand = jnp.concatenate([xb[:, lo:S], z4], axis=1)
            else:
                cand = xb[:, lo : lo + W]
            term = cand * m_ref[0, k : k + 1, :W]
            acc = term if acc is None else acc + term
        o_ref[...] = acc

    y_tc = pl.pallas_call(
        tc_select_body,
        grid=(TC_ROWS // _TC_BLK,),
        in_specs=[
            pl.BlockSpec((_TC_BLK, S), lambda i: (i, 0)),
            pl.BlockSpec(
                (1, _KERNEL_SIZE, WP), lambda i: (i * _TC_BLK // F, 0, 0)
            ),
        ],
        out_specs=pl.BlockSpec((_TC_BLK, W), lambda i: (i, 0)),
        out_shape=jax.ShapeDtypeStruct((R, W), x.dtype),
    )(x2, selm)

    y_sc = sc_gather_blocks(x3, src_pad, mask)

    BPG = _TC_BLK // 8

    def fold_body(ysc_ref, yfull_ref, o_ref):
        del yfull_ref
        yb = ysc_ref[...]
        yb = yb.transpose(0, 2, 1, 3).reshape(_TC_BLK, WP)
        o_ref[...] = yb[:, :W]

    out = pl.pallas_call(
        fold_body,
        grid=(_SC_ROWS // _TC_BLK,),
        in_specs=[
            pl.BlockSpec((BPG, CT, 8, 128), lambda i: (i, 0, 0, 0)),
            pl.BlockSpec((_TC_BLK, W), lambda i: (0, 0)),
        ],
        out_specs=pl.BlockSpec(
            (_TC_BLK, W), lambda i: (i + TC_ROWS // _TC_BLK, 0)
        ),
        out_shape=jax.ShapeDtypeStruct((R, W), x.dtype),
        input_output_aliases={1: 0},
    )(y_sc, y_tc)

    return out.reshape(B, F, W)


# speedup vs baseline: 9.5458x; 1.0792x over previous
import dataclasses
import functools

import jax
import jax.numpy as jnp
import numpy as np
from jax import lax
from jax.experimental import pallas as pl
from jax.experimental.pallas import tpu as pltpu
from jax.experimental.pallas import tpu_sc as plsc

_KERNEL_SIZE = 5
_PADDING = 4

_NC = 2
_NS = 16
_LANES = 16

_SC_ROWS = 1024
_TC_BLK = 256


def _compiler_params():
    cp = pltpu.CompilerParams(use_tc_tiling_on_sc=True)
    if "needs_layout_passes" in pltpu.CompilerParams.__dataclass_fields__:
        cp = dataclasses.replace(cp, needs_layout_passes=False)
    return cp


@functools.lru_cache(maxsize=None)
def _tables(B, W, S, WP):
    with jax.ensure_compile_time_eval():
        idx = np.asarray(
            jax.random.randint(
                jax.random.key(1), (B, W), 0, _KERNEL_SIZE, dtype=jnp.int32
            )
        )
    w_ar = np.arange(W, dtype=np.int32)
    src = idx * W + w_ar[None, :] - _PADDING
    valid = (src >= 0) & (src < S)
    src_pad = np.zeros((B, WP), np.int32)
    src_pad[:, :W] = np.where(valid, src, 0)
    mask = np.zeros((B, WP), np.float32)
    mask[:, :W] = valid.astype(np.float32)
    sel = idx[:, None, :] == np.arange(_KERNEL_SIZE)[None, :, None]
    selm = np.zeros((B, _KERNEL_SIZE, WP), np.float32)
    selm[:, :, :W] = (sel & valid[:, None, :]).astype(np.float32)
    return src_pad, mask, selm


def kernel(x):
    B, F, S = x.shape
    W = (S + 2 * _PADDING) // _KERNEL_SIZE
    WP = ((W + 127) // 128) * 128
    CT = WP // 128

    src_pad, mask, selm = _tables(B, W, S, WP)
    src_pad, mask, selm = (
        jnp.asarray(src_pad), jnp.asarray(mask), jnp.asarray(selm),
    )

    NW = _NC * _NS
    R = B * F
    NBLK = R // 8
    TC_ROWS = R - _SC_ROWS
    SC_BLKS = _SC_ROWS // 8
    SC_OFF_BLK = TC_ROWS // 8
    blocks_per_tile = SC_BLKS // NW
    x3 = x.reshape(NBLK, 8, S)
    x2 = x.reshape(R, S)

    mesh = plsc.VectorSubcoreMesh(core_axis_name="c", subcore_axis_name="s")

    @functools.partial(
        pl.kernel,
        out_type=jax.ShapeDtypeStruct((_SC_ROWS, WP), x.dtype),
        mesh=mesh,
        compiler_params=_compiler_params(),
        scratch_types=[
            pltpu.VMEM((B, WP), jnp.int32),
            pltpu.VMEM((B, WP), jnp.float32),
            pltpu.VMEM((8, S), jnp.float32),
            pltpu.VMEM((8, WP), jnp.float32),
            pltpu.VMEM((8, WP), jnp.float32),
            pltpu.SemaphoreType.DMA,
            pltpu.SemaphoreType.DMA,
        ],
    )
    def sc_gather_blocks(x_hbm, src_hbm, msk_hbm, o_hbm, src_v, msk_v, ib,
                         ob0, ob1, osem0, osem1):
        wid = lax.axis_index("s") * _NC + lax.axis_index("c")
        base = SC_OFF_BLK + wid * blocks_per_tile
        bt = (base * 8) // F
        pltpu.sync_copy(src_hbm, src_v)
        pltpu.sync_copy(msk_hbm, msk_v)

        def out_copy(t, obuf, sem):
            return pltpu.make_async_copy(
                obuf,
                o_hbm.at[pl.ds((base - SC_OFF_BLK + t) * 8, 8), :],
                sem,
            )

        def do_block(t, obuf):
            pltpu.sync_copy(x_hbm.at[base + t], ib)

            @pl.loop(0, CT)
            def _(c):
                for u in range(8):
                    iv = src_v[bt, pl.ds(c * 128 + u * 16, _LANES)]
                    mv = msk_v[bt, pl.ds(c * 128 + u * 16, _LANES)]
                    for fs in range(8):
                        rowv = jnp.full((_LANES,), fs, jnp.int32)
                        g = plsc.load_gather(ib, [rowv, iv])
                        obuf[fs, pl.ds(c * 128 + u * 16, _LANES)] = g * mv

        @pl.loop(0, blocks_per_tile, step=2)
        def _(t):
            @pl.when(t >= 2)
            def _():
                out_copy(t - 2, ob0, osem0).wait()

            do_block(t, ob0)
            out_copy(t, ob0, osem0).start()

            @pl.when(t >= 2)
            def _():
                out_copy(t - 1, ob1, osem1).wait()

            do_block(t + 1, ob1)
            out_copy(t + 1, ob1, osem1).start()

        out_copy(blocks_per_tile - 2, ob0, osem0).wait()
        out_copy(blocks_per_tile - 1, ob1, osem1).wait()

    def tc_select_body(x_ref, m_ref, o_ref):
        xb = x_ref[...]
        z4 = jnp.zeros((_TC_BLK, _PADDING), jnp.float32)
        acc = None
        for k in range(_KERNEL_SIZE):
            lo = k * W - _PADDING
            if lo < 0:
                cand = jnp.concatenate([z4, xb[:, : W + lo]], axis=1)
            elif lo + W > S:
                cand = jnp.concatenate([xb[:, lo:S], z4], axis=1)
            else:
                cand = xb[:, lo : lo + W]
            term = cand * m_ref[0, k : k + 1, :W]
            acc = term if acc is None else acc + term
        o_ref[...] = acc

    y_tc = pl.pallas_call(
        tc_select_body,
        grid=(TC_ROWS // _TC_BLK,),
        in_specs=[
            pl.BlockSpec((_TC_BLK, S), lambda i: (i, 0)),
            pl.BlockSpec(
                (1, _KERNEL_SIZE, WP), lambda i: (i * _TC_BLK // F, 0, 0)
            ),
        ],
        out_specs=pl.BlockSpec((_TC_BLK, W), lambda i: (i, 0)),
        out_shape=jax.ShapeDtypeStruct((R, W), x.dtype),
    )(x2, selm)

    y_sc = sc_gather_blocks(x3, src_pad, mask)

    def fold_body(ysc_ref, yfull_ref, o_ref):
        del yfull_ref
        o_ref[...] = ysc_ref[...][:, :W]

    out = pl.pallas_call(
        fold_body,
        grid=(_SC_ROWS // _TC_BLK,),
        in_specs=[
            pl.BlockSpec((_TC_BLK, WP), lambda i: (i, 0)),
            pl.BlockSpec((_TC_BLK, W), lambda i: (0, 0)),
        ],
        out_specs=pl.BlockSpec(
            (_TC_BLK, W), lambda i: (i + TC_ROWS // _TC_BLK, 0)
        ),
        out_shape=jax.ShapeDtypeStruct((R, W), x.dtype),
        input_output_aliases={1: 0},
    )(y_sc, y_tc)

    return out.reshape(B, F, W)
